# baseline (speedup 1.0000x reference)
import jax
import jax.numpy as jnp
from jax import lax
from jax.experimental import pallas as pl
from jax.experimental.pallas import tpu as pltpu
from jax.experimental.pallas import tpu_sc as plsc

VOCAB = 1_000_000
SEQ = 200
D = 64
BATCH = 4096

NC, NS = 2, 16
NW = NC * NS
B_TOTAL = BATCH * SEQ
B_PER_W = B_TOTAL // NW
CHUNK = 128
N_CHUNKS = B_PER_W // CHUNK
POS_TILE = 3200
NFILL = POS_TILE // CHUNK
L = 16


def _emb_kernel(idx_hbm, table_hbm, pos_hbm, out_hbm,
                idx_v, idx2_v, par_v, pair_v, out_v,
                in_s0, in_s1, g_s0, g_s1, o_s0, o_s1):
    in_sem = (in_s0, in_s1)
    g_sem = (g_s0, g_s1)
    out_sem = (o_s0, o_s1)
    wid = lax.axis_index("s") * NC + lax.axis_index("c")
    base = wid * B_PER_W

    def in_descs(g, s):
        row0 = base + g * CHUNK
        pos0 = lax.rem(g, NFILL) * CHUNK
        return (
            pltpu.make_async_copy(
                idx_hbm.at[pl.ds(row0, CHUNK)], idx_v.at[s], in_sem[s]),
            pltpu.make_async_copy(
                pos_hbm.at[pl.ds(pos0, CHUNK)], out_v.at[s], in_sem[s]),
        )

    def gather_desc(s):
        return pltpu.make_async_copy(
            table_hbm.at[idx2_v.at[s]], pair_v.at[s], g_sem[s])

    def out_desc(g, s):
        row0 = base + g * CHUNK
        return pltpu.make_async_copy(
            out_v.at[s], out_hbm.at[pl.ds(row0, CHUNK)], out_sem[s])

    def split_indices(s):
        def vbody(v, _):
            x = idx_v[s, pl.ds(v * L, L)]
            idx2_v[s, pl.ds(v * L, L)] = lax.shift_right_logical(x, 1)
            par_v[s, pl.ds(v * L, L)] = (x & 1) * D
            return _
        lax.fori_loop(0, CHUNK // L, vbody, None)

    def add_halves(s):
        def gbody(v, _):
            par16 = par_v[s, pl.ds(v * L, L)]
            for lane in range(L):
                r = v * L + lane
                p = par16[lane]
                for k in range(D // L):
                    plsc.addupdate(
                        out_v.at[s, r, pl.ds(k * L, L)],
                        pair_v[s, r, pl.ds(p + k * L, L)],
                    )
            return _
        lax.fori_loop(0, CHUNK // L, gbody, None)

    def process(g, s):
        for d in in_descs(g, s):
            d.wait()
        split_indices(s)
        gather_desc(s).start()

        o = 1 - s

        @pl.when(g > 0)
        def _():
            out_desc(g - 1, o).wait()

        @pl.when(g + 1 < N_CHUNKS)
        def _():
            for d in in_descs(g + 1, o):
                d.start()

        gather_desc(s).wait()
        add_halves(s)
        out_desc(g, s).start()

    for d in in_descs(0, 0):
        d.start()

    def body(i, _):
        process(2 * i, 0)
        process(2 * i + 1, 1)
        return _

    lax.fori_loop(0, N_CHUNKS // 2, body, None)
    out_desc(N_CHUNKS - 1, 1).wait()


@jax.jit
def _embed(idx_flat, table_pairs, pos_tiled):
    mesh = plsc.VectorSubcoreMesh(
        core_axis_name="c", subcore_axis_name="s", num_cores=NC, num_subcores=NS
    )
    fn = pl.kernel(
        _emb_kernel,
        out_type=jax.ShapeDtypeStruct((B_TOTAL, D), jnp.float32),
        mesh=mesh,
        scratch_types=[
            pltpu.VMEM((2, CHUNK), jnp.int32),
            pltpu.VMEM((2, CHUNK), jnp.int32),
            pltpu.VMEM((2, CHUNK), jnp.int32),
            pltpu.VMEM((2, CHUNK, 2 * D), jnp.float32),
            pltpu.VMEM((2, CHUNK, D), jnp.float32),
            pltpu.SemaphoreType.DMA,
            pltpu.SemaphoreType.DMA,
            pltpu.SemaphoreType.DMA,
            pltpu.SemaphoreType.DMA,
            pltpu.SemaphoreType.DMA,
            pltpu.SemaphoreType.DMA,
        ],
    )
    return fn(idx_flat, table_pairs, pos_tiled)


def kernel(inputs, token_table, pos_table):
    idx_flat = inputs.astype(jnp.int32).reshape(B_TOTAL)
    table_pairs = token_table.astype(jnp.float32).reshape(VOCAB // 2, 2 * D)
    pos_tiled = jnp.tile(pos_table.astype(jnp.float32), (POS_TILE // SEQ, 1))
    out = _embed(idx_flat, table_pairs, pos_tiled)
    return out.reshape(BATCH, SEQ, D)
